# Initial kernel scaffold; baseline (speedup 1.0000x reference)
import jax
import jax.numpy as jnp
from jax.experimental import pallas as pl


def kernel(features, edge_index, W0, b0, W1, b1, W2, b2, W3, b3, eps):
    raise NotImplementedError("write your pallas kernel here")


# speedup vs baseline: 3.9631x; 3.9631x over previous
import functools

import jax
import jax.numpy as jnp
from jax import lax
from jax.experimental import pallas as pl
from jax.experimental.pallas import tpu as pltpu
from jax.experimental.pallas import tpu_sc as plsc

N_NODES = 10000
N_PAD = 10240
JUNK_ROW = N_NODES
BLK = 512
CH = 128
N_TILES = 32


def _sc_agg(D, e_pad):
  mesh = plsc.VectorSubcoreMesh(core_axis_name="c", subcore_axis_name="s")
  rows_per_tile = N_PAD // 16
  ept = e_pad // N_TILES
  n_chunks = ept // CH

  @functools.partial(
      pl.kernel,
      mesh=mesh,
      compiler_params=pltpu.CompilerParams(use_tc_tiling_on_sc=False),
      out_type=jax.ShapeDtypeStruct((2, N_PAD, D), jnp.float32),
      scratch_types=[
          pltpu.VMEM_SHARED((N_PAD, D), jnp.float32),
          pltpu.VMEM((CH,), jnp.int32),
          pltpu.VMEM((CH,), jnp.int32),
          pltpu.VMEM((CH, D), jnp.float32),
          pltpu.VMEM((16, D), jnp.float32),
          pltpu.SemaphoreType.DMA,
      ],
  )
  def k(y_hbm, src_hbm, dst_hbm, out_hbm, acc, src_v, dst_v, rows_v, zblk, sem):
    c = lax.axis_index("c")
    s = lax.axis_index("s")
    wid = s * 2 + c
    z16 = jnp.zeros((16,), jnp.float32)
    for r in range(16):
      for q in range(D // 16):
        zblk[r, pl.ds(q * 16, 16)] = z16
    row0 = s * rows_per_tile

    def zero_body(j, carry):
      pltpu.sync_copy(zblk, acc.at[pl.ds(row0 + j * 16, 16)])
      return carry

    lax.fori_loop(0, rows_per_tile // 16, zero_body, 0)
    plsc.subcore_barrier()

    ebase = wid * ept

    def body(i, carry):
      off = ebase + i * CH
      pltpu.sync_copy(src_hbm.at[pl.ds(off, CH)], src_v)
      pltpu.sync_copy(dst_hbm.at[pl.ds(off, CH)], dst_v)
      pltpu.async_copy(y_hbm.at[src_v], rows_v, sem).wait()
      pltpu.sync_copy(rows_v, acc.at[dst_v], add=True)
      return carry

    lax.fori_loop(0, n_chunks, body, 0)
    plsc.subcore_barrier()
    pltpu.sync_copy(acc.at[pl.ds(row0, rows_per_tile)],
                    out_hbm.at[c, pl.ds(row0, rows_per_tile)])

  return k


def _mm_body(x_ref, w_ref, o_ref):
  o_ref[...] = jnp.dot(x_ref[...], w_ref[...], preferred_element_type=jnp.float32)


def _tc_matmul(x, w):
  n, kdim = x.shape
  dout = w.shape[1]
  return pl.pallas_call(
      _mm_body,
      grid=(n // BLK,),
      in_specs=[
          pl.BlockSpec((BLK, kdim), lambda i: (i, 0)),
          pl.BlockSpec((kdim, dout), lambda i: (0, 0)),
      ],
      out_specs=pl.BlockSpec((BLK, dout), lambda i: (i, 0)),
      out_shape=jax.ShapeDtypeStruct((n, dout), jnp.float32),
  )(x, w)


def _comb_mm_body(eps_ref, y_ref, p0_ref, p1_ref, d0_ref, d1_ref, b_ref, w_ref,
                  o_ref):
  deg = jnp.maximum(d0_ref[...] + d1_ref[...], 1.0)
  inv = 1.0 / deg[:, 0:1]
  agg = (p0_ref[...] + p1_ref[...]) * inv
  h = (1.0 + eps_ref[0, 0]) * y_ref[...] + agg + b_ref[...]
  h = jnp.maximum(h, 0.0)
  o_ref[...] = jnp.dot(h, w_ref[...], preferred_element_type=jnp.float32)


def _comb_final_body(eps_ref, y_ref, p0_ref, p1_ref, d0_ref, d1_ref, b_ref,
                     o_ref):
  deg = jnp.maximum(d0_ref[...] + d1_ref[...], 1.0)
  inv = 1.0 / deg[:, 0:1]
  agg = (p0_ref[...] + p1_ref[...]) * inv
  o_ref[...] = (1.0 + eps_ref[0, 0]) * y_ref[...] + agg + b_ref[...]


def _combine_mm(eps_i, y, p0, p1, d0, d1, b, w):
  n, dy = y.shape
  dout = w.shape[1]
  return pl.pallas_call(
      _comb_mm_body,
      grid=(n // BLK,),
      in_specs=[
          pl.BlockSpec(memory_space=pltpu.SMEM),
          pl.BlockSpec((BLK, dy), lambda i: (i, 0)),
          pl.BlockSpec((BLK, dy), lambda i: (i, 0)),
          pl.BlockSpec((BLK, dy), lambda i: (i, 0)),
          pl.BlockSpec((BLK, 16), lambda i: (i, 0)),
          pl.BlockSpec((BLK, 16), lambda i: (i, 0)),
          pl.BlockSpec((1, dy), lambda i: (0, 0)),
          pl.BlockSpec((dy, dout), lambda i: (0, 0)),
      ],
      out_specs=pl.BlockSpec((BLK, dout), lambda i: (i, 0)),
      out_shape=jax.ShapeDtypeStruct((n, dout), jnp.float32),
  )(eps_i.reshape(1, 1), y, p0, p1, d0, d1, b.reshape(1, dy), w)


def _combine_final(eps_i, y, p0, p1, d0, d1, b):
  n, dy = y.shape
  return pl.pallas_call(
      _comb_final_body,
      grid=(n // BLK,),
      in_specs=[
          pl.BlockSpec(memory_space=pltpu.SMEM),
          pl.BlockSpec((BLK, dy), lambda i: (i, 0)),
          pl.BlockSpec((BLK, dy), lambda i: (i, 0)),
          pl.BlockSpec((BLK, dy), lambda i: (i, 0)),
          pl.BlockSpec((BLK, 16), lambda i: (i, 0)),
          pl.BlockSpec((BLK, 16), lambda i: (i, 0)),
          pl.BlockSpec((1, dy), lambda i: (0, 0)),
      ],
      out_specs=pl.BlockSpec((BLK, dy), lambda i: (i, 0)),
      out_shape=jax.ShapeDtypeStruct((n, dy), jnp.float32),
  )(eps_i.reshape(1, 1), y, p0, p1, d0, d1, b.reshape(1, dy))


def kernel(features, edge_index, W0, b0, W1, b1, W2, b2, W3, b3, eps):
  E = edge_index.shape[1]
  src, dst = edge_index[0], edge_index[1]
  gran = N_TILES * CH
  e_pad = ((E + gran - 1) // gran) * gran
  padn = e_pad - E
  if padn:
    src = jnp.concatenate([src, jnp.zeros((padn,), jnp.int32)])
    dst = jnp.concatenate([dst, jnp.full((padn,), JUNK_ROW, jnp.int32)])

  xp = jnp.pad(features, ((0, N_PAD - features.shape[0]), (0, 0)))

  y0 = _tc_matmul(xp, W0)
  y0c = jnp.concatenate([y0, jnp.ones((N_PAD, 16), jnp.float32)], axis=1)

  agg0 = _sc_agg(144, e_pad)(y0c, src, dst)
  p0, p1 = agg0[0, :, :128], agg0[1, :, :128]
  d0, d1 = agg0[0, :, 128:], agg0[1, :, 128:]

  y1 = _combine_mm(eps[0], y0, p0, p1, d0, d1, b0, W1)
  agg1 = _sc_agg(128, e_pad)(y1, src, dst)
  y2 = _combine_mm(eps[1], y1, agg1[0], agg1[1], d0, d1, b1, W2)
  agg2 = _sc_agg(128, e_pad)(y2, src, dst)
  W3p = jnp.pad(W3, ((0, 0), (0, 8)))
  y3 = _combine_mm(eps[2], y2, agg2[0], agg2[1], d0, d1, b2, W3p)
  agg3 = _sc_agg(48, e_pad)(y3, src, dst)
  b3p = jnp.pad(b3, (0, 8))
  out48 = _combine_final(eps[3], y3, agg3[0], agg3[1], d0, d1, b3p)
  return out48[:N_NODES, :40]


# speedup vs baseline: 4.0369x; 1.0186x over previous
import functools

import jax
import jax.numpy as jnp
from jax import lax
from jax.experimental import pallas as pl
from jax.experimental.pallas import tpu as pltpu
from jax.experimental.pallas import tpu_sc as plsc

N_NODES = 10000
N_PAD = 10240
JUNK_ROW = N_NODES
BLK = 512
CH = 128
NB = 4
N_TILES = 32
RPT = N_PAD // 16

_MESH = dict(core_axis_name="c", subcore_axis_name="s")
_SC_PARAMS = dict(
    compiler_params=pltpu.CompilerParams(use_tc_tiling_on_sc=False))


def _fill(ref, rows, width, vec):
  for r in range(rows):
    for q in range(width // 16):
      ref[r, pl.ds(q * 16, 16)] = vec


def _sc_deg(nch):
  @functools.partial(
      pl.kernel,
      mesh=plsc.VectorSubcoreMesh(**_MESH),
      out_type=jax.ShapeDtypeStruct((2, N_PAD, 16), jnp.float32),
      scratch_types=[
          pltpu.VMEM_SHARED((N_PAD, 16), jnp.float32),
          pltpu.VMEM((nch, CH), jnp.int32),
          pltpu.VMEM((CH, 16), jnp.float32),
          pltpu.VMEM((64, 16), jnp.float32),
          pltpu.SemaphoreType.DMA,
      ],
      **_SC_PARAMS,
  )
  def k(dst_hbm, out_hbm, acc, dst_v, ones_v, zblk, sem):
    c = lax.axis_index("c")
    s = lax.axis_index("s")
    wid = s * 2 + c
    _fill(ones_v, CH, 16, jnp.ones((16,), jnp.float32))
    _fill(zblk, 64, 16, jnp.zeros((16,), jnp.float32))
    row0 = s * RPT

    def zbody(j, carry):
      pltpu.sync_copy(zblk, acc.at[pl.ds(row0 + j * 64, 64)])
      return carry

    lax.fori_loop(0, RPT // 64, zbody, 0)
    pltpu.sync_copy(dst_hbm.at[pl.ds(wid * nch, nch)], dst_v)
    plsc.subcore_barrier()

    def group(g, carry):
      j0 = g * NB
      hs = [pltpu.async_copy(ones_v, acc.at[dst_v.at[j0 + b]], sem, add=True)
            for b in range(NB)]
      for h in hs:
        h.wait()
      return carry

    lax.fori_loop(0, nch // NB, group, 0)
    plsc.subcore_barrier()
    pltpu.sync_copy(acc.at[pl.ds(row0, RPT)], out_hbm.at[c, pl.ds(row0, RPT)])

  return k


def _sc_split(nch):
  D = 64

  @functools.partial(
      pl.kernel,
      mesh=plsc.VectorSubcoreMesh(**_MESH),
      out_type=jax.ShapeDtypeStruct((2, N_PAD, D), jnp.float32),
      scratch_types=[
          pltpu.VMEM_SHARED((N_PAD, D), jnp.float32),
          pltpu.VMEM((nch, CH), jnp.int32),
          pltpu.VMEM((nch, CH), jnp.int32),
      ] + [pltpu.VMEM((CH, D), jnp.float32) for _ in range(NB)] + [
          pltpu.VMEM((16, D), jnp.float32),
          pltpu.SemaphoreType.DMA,
          pltpu.SemaphoreType.DMA,
      ],
      **_SC_PARAMS,
  )
  def k(ylo_hbm, yhi_hbm, src_hbm, dst_hbm, out_hbm, acc, src_v, dst_v,
        r0, r1, r2, r3, zblk, gsem, ssem):
    rows = [r0, r1, r2, r3]
    c = lax.axis_index("c")
    s = lax.axis_index("s")
    _fill(zblk, 16, D, jnp.zeros((16,), jnp.float32))
    row0 = s * RPT

    def zbody(j, carry):
      pltpu.sync_copy(zblk, acc.at[pl.ds(row0 + j * 16, 16)])
      return carry

    lax.fori_loop(0, RPT // 16, zbody, 0)
    pltpu.sync_copy(src_hbm.at[pl.ds(s * nch, nch)], src_v)
    pltpu.sync_copy(dst_hbm.at[pl.ds(s * nch, nch)], dst_v)
    plsc.subcore_barrier()

    def run(y_hbm):
      def group(g, carry):
        j0 = g * NB
        ghs = [pltpu.async_copy(y_hbm.at[src_v.at[j0 + b]], rows[b], gsem)
               for b in range(NB)]
        shs = []
        for b in range(NB):
          ghs[b].wait()
          shs.append(pltpu.async_copy(rows[b], acc.at[dst_v.at[j0 + b]], ssem,
                                      add=True))
        for h in shs:
          h.wait()
        return carry

      lax.fori_loop(0, nch // NB, group, 0)

    @pl.when(c == 0)
    def _():
      run(ylo_hbm)

    @pl.when(c == 1)
    def _():
      run(yhi_hbm)

    plsc.subcore_barrier()
    pltpu.sync_copy(acc.at[pl.ds(row0, RPT)], out_hbm.at[c, pl.ds(row0, RPT)])

  return k


def _sc_agg48(nch):
  D = 48

  @functools.partial(
      pl.kernel,
      mesh=plsc.VectorSubcoreMesh(**_MESH),
      out_type=jax.ShapeDtypeStruct((2, N_PAD, D), jnp.float32),
      scratch_types=[
          pltpu.VMEM_SHARED((N_PAD, D), jnp.float32),
          pltpu.VMEM((nch, CH), jnp.int32),
          pltpu.VMEM((nch, CH), jnp.int32),
      ] + [pltpu.VMEM((CH, D), jnp.float32) for _ in range(NB)] + [
          pltpu.VMEM((16, D), jnp.float32),
          pltpu.SemaphoreType.DMA,
          pltpu.SemaphoreType.DMA,
      ],
      **_SC_PARAMS,
  )
  def k(y_hbm, src_hbm, dst_hbm, out_hbm, acc, src_v, dst_v,
        r0, r1, r2, r3, zblk, gsem, ssem):
    rows = [r0, r1, r2, r3]
    c = lax.axis_index("c")
    s = lax.axis_index("s")
    wid = s * 2 + c
    _fill(zblk, 16, D, jnp.zeros((16,), jnp.float32))
    row0 = s * RPT

    def zbody(j, carry):
      pltpu.sync_copy(zblk, acc.at[pl.ds(row0 + j * 16, 16)])
      return carry

    lax.fori_loop(0, RPT // 16, zbody, 0)
    pltpu.sync_copy(src_hbm.at[pl.ds(wid * nch, nch)], src_v)
    pltpu.sync_copy(dst_hbm.at[pl.ds(wid * nch, nch)], dst_v)
    plsc.subcore_barrier()

    def group(g, carry):
      j0 = g * NB
      ghs = [pltpu.async_copy(y_hbm.at[src_v.at[j0 + b]], rows[b], gsem)
             for b in range(NB)]
      shs = []
      for b in range(NB):
        ghs[b].wait()
        shs.append(pltpu.async_copy(rows[b], acc.at[dst_v.at[j0 + b]], ssem,
                                    add=True))
      for h in shs:
        h.wait()
      return carry

    lax.fori_loop(0, nch // NB, group, 0)
    plsc.subcore_barrier()
    pltpu.sync_copy(acc.at[pl.ds(row0, RPT)], out_hbm.at[c, pl.ds(row0, RPT)])

  return k


def _grid_specs(n_out, specs, out_specs):
  return dict(
      grid=(N_PAD // BLK,),
      in_specs=specs,
      out_specs=out_specs,
      out_shape=[jax.ShapeDtypeStruct((N_PAD, d), jnp.float32)
                 for d in n_out],
  )


def _mm2_body(x_ref, wlo_ref, whi_ref, olo_ref, ohi_ref):
  x = x_ref[...]
  olo_ref[...] = jnp.dot(x, wlo_ref[...], preferred_element_type=jnp.float32)
  ohi_ref[...] = jnp.dot(x, whi_ref[...], preferred_element_type=jnp.float32)


def _tc_matmul2(x, w):
  kdim = x.shape[1]
  return pl.pallas_call(
      _mm2_body,
      **_grid_specs(
          (64, 64),
          [
              pl.BlockSpec((BLK, kdim), lambda i: (i, 0)),
              pl.BlockSpec((kdim, 64), lambda i: (0, 0)),
              pl.BlockSpec((kdim, 64), lambda i: (0, 0)),
          ],
          [pl.BlockSpec((BLK, 64), lambda i: (i, 0))] * 2,
      ),
  )(x, w[:, :64], w[:, 64:])


def _inv_deg(d0, d1):
  deg = jnp.maximum(d0[0] + d1[0], 1.0)
  return 1.0 / deg[:, 0:1]


def _comb_split_body(eps_ref, ylo_ref, yhi_ref, plo_ref, phi_ref, d0_ref,
                     d1_ref, blo_ref, bhi_ref, wlo_ref, whi_ref,
                     olo_ref, ohi_ref):
  inv = _inv_deg(d0_ref[...], d1_ref[...])
  sc = 1.0 + eps_ref[0, 0]
  hlo = jnp.maximum(sc * ylo_ref[...] + plo_ref[0] * inv + blo_ref[...], 0.0)
  hhi = jnp.maximum(sc * yhi_ref[...] + phi_ref[0] * inv + bhi_ref[...], 0.0)
  h = jnp.concatenate([hlo, hhi], axis=1)
  olo_ref[...] = jnp.dot(h, wlo_ref[...], preferred_element_type=jnp.float32)
  ohi_ref[...] = jnp.dot(h, whi_ref[...], preferred_element_type=jnp.float32)


def _comb_to48_body(eps_ref, ylo_ref, yhi_ref, plo_ref, phi_ref, d0_ref,
                    d1_ref, blo_ref, bhi_ref, w_ref, o_ref):
  inv = _inv_deg(d0_ref[...], d1_ref[...])
  sc = 1.0 + eps_ref[0, 0]
  hlo = jnp.maximum(sc * ylo_ref[...] + plo_ref[0] * inv + blo_ref[...], 0.0)
  hhi = jnp.maximum(sc * yhi_ref[...] + phi_ref[0] * inv + bhi_ref[...], 0.0)
  h = jnp.concatenate([hlo, hhi], axis=1)
  o_ref[...] = jnp.dot(h, w_ref[...], preferred_element_type=jnp.float32)


def _comb_final_body(eps_ref, y_ref, p0_ref, p1_ref, d0_ref, d1_ref, b_ref,
                     o_ref):
  inv = _inv_deg(d0_ref[...], d1_ref[...])
  agg = (p0_ref[0] + p1_ref[0]) * inv
  o_ref[...] = (1.0 + eps_ref[0, 0]) * y_ref[...] + agg + b_ref[...]


_HALF = lambda half: (lambda i, h=half: (h, i, 0))
_DSPECS = [pl.BlockSpec((1, BLK, 16), _HALF(0)),
           pl.BlockSpec((1, BLK, 16), _HALF(1))]


def _split_in_specs():
  return [
      pl.BlockSpec(memory_space=pltpu.SMEM),
      pl.BlockSpec((BLK, 64), lambda i: (i, 0)),
      pl.BlockSpec((BLK, 64), lambda i: (i, 0)),
      pl.BlockSpec((1, BLK, 64), _HALF(0)),
      pl.BlockSpec((1, BLK, 64), _HALF(1)),
      *_DSPECS,
      pl.BlockSpec((1, 64), lambda i: (0, 0)),
      pl.BlockSpec((1, 64), lambda i: (0, 0)),
  ]


def _combine_split(eps_i, ylo, yhi, p, aggd, b, w):
  return pl.pallas_call(
      _comb_split_body,
      **_grid_specs(
          (64, 64),
          _split_in_specs() + [
              pl.BlockSpec((128, 64), lambda i: (0, 0)),
              pl.BlockSpec((128, 64), lambda i: (0, 0)),
          ],
          [pl.BlockSpec((BLK, 64), lambda i: (i, 0))] * 2,
      ),
  )(eps_i.reshape(1, 1), ylo, yhi, p, p, aggd, aggd,
    b[:64].reshape(1, 64), b[64:].reshape(1, 64), w[:, :64], w[:, 64:])


def _combine_to48(eps_i, ylo, yhi, p, aggd, b, w48):
  return pl.pallas_call(
      _comb_to48_body,
      **_grid_specs(
          (48,),
          _split_in_specs() + [pl.BlockSpec((128, 48), lambda i: (0, 0))],
          [pl.BlockSpec((BLK, 48), lambda i: (i, 0))],
      ),
  )(eps_i.reshape(1, 1), ylo, yhi, p, p, aggd, aggd,
    b[:64].reshape(1, 64), b[64:].reshape(1, 64), w48)[0]


def _combine_final(eps_i, y, p, aggd, b):
  return pl.pallas_call(
      _comb_final_body,
      **_grid_specs(
          (48,),
          [
              pl.BlockSpec(memory_space=pltpu.SMEM),
              pl.BlockSpec((BLK, 48), lambda i: (i, 0)),
              pl.BlockSpec((1, BLK, 48), _HALF(0)),
              pl.BlockSpec((1, BLK, 48), _HALF(1)),
              *_DSPECS,
              pl.BlockSpec((1, 48), lambda i: (0, 0)),
          ],
          [pl.BlockSpec((BLK, 48), lambda i: (i, 0))],
      ),
  )(eps_i.reshape(1, 1), y, p, p, aggd, aggd, b.reshape(1, 48))[0]


def kernel(features, edge_index, W0, b0, W1, b1, W2, b2, W3, b3, eps):
  E = edge_index.shape[1]
  src, dst = edge_index[0], edge_index[1]
  gran = N_TILES * CH * NB
  e_pad = ((E + gran - 1) // gran) * gran
  padn = e_pad - E
  if padn:
    src = jnp.concatenate([src, jnp.zeros((padn,), jnp.int32)])
    dst = jnp.concatenate([dst, jnp.full((padn,), JUNK_ROW, jnp.int32)])
  src2 = src.reshape(-1, CH)
  dst2 = dst.reshape(-1, CH)
  nch32 = e_pad // (N_TILES * CH)
  nch16 = 2 * nch32

  aggd = _sc_deg(nch32)(dst2)
  y0l, y0h = _tc_matmul2(features, W0)
  p0 = _sc_split(nch16)(y0l, y0h, src2, dst2)
  y1l, y1h = _combine_split(eps[0], y0l, y0h, p0, aggd, b0, W1)
  p1 = _sc_split(nch16)(y1l, y1h, src2, dst2)
  y2l, y2h = _combine_split(eps[1], y1l, y1h, p1, aggd, b1, W2)
  p2 = _sc_split(nch16)(y2l, y2h, src2, dst2)
  W3p = jnp.pad(W3, ((0, 0), (0, 8)))
  y3 = _combine_to48(eps[2], y2l, y2h, p2, aggd, b2, W3p)
  p3 = _sc_agg48(nch32)(y3, src2, dst2)
  out48 = _combine_final(eps[3], y3, p3, aggd, jnp.pad(b3, (0, 8)))
  return out48[:N_NODES, :40]
